# baseline (speedup 1.0000x reference)
import jax
import jax.numpy as jnp
from jax import lax
from jax.experimental import pallas as pl
from jax.experimental.pallas import tpu as pltpu
from jax.experimental.pallas import tpu_sc as plsc

DIM = 1024
E = 8
HIDDEN = 2048
N_TOK = 4096
BLK = 128
A = N_TOK * 2
G = A + E * BLK
NB = G // BLK
WL = 128

_NC, _NS = 2, 16
_NW = _NC * _NS


def _router_body(rs_ref, x_ref, gw_ref, i0_ref, i1_ref, w0_ref, w1_ref):
    xf = x_ref[...]
    gw = gw_ref[...]
    logits = lax.dot_general(gw, xf, (((1,), (1,)), ((), ())),
                             preferred_element_type=jnp.float32)
    scaled = logits * rs_ref[0]
    iota = lax.broadcasted_iota(jnp.int32, scaled.shape, 0)
    m0 = jnp.max(scaled, axis=0, keepdims=True)
    i0 = jnp.min(jnp.where(scaled == m0, iota, E), axis=0, keepdims=True)
    masked = jnp.where(iota == i0, -jnp.inf, scaled)
    m1 = jnp.max(masked, axis=0, keepdims=True)
    i1 = jnp.min(jnp.where(masked == m1, iota, E), axis=0, keepdims=True)
    e1 = jnp.exp(m1 - m0)
    w0 = 1.0 / (1.0 + e1)
    w1 = e1 / (1.0 + e1)
    i0_ref[...] = i0
    i1_ref[...] = i1
    w0_ref[...] = w0
    w1_ref[...] = w1


def _router(xf, gate_w, router_scale):
    out = pl.pallas_call(
        _router_body,
        in_specs=[
            pl.BlockSpec(memory_space=pltpu.SMEM),
            pl.BlockSpec(memory_space=pltpu.VMEM),
            pl.BlockSpec(memory_space=pltpu.VMEM),
        ],
        out_specs=[pl.BlockSpec(memory_space=pltpu.VMEM)] * 4,
        out_shape=[
            jax.ShapeDtypeStruct((1, N_TOK), jnp.int32),
            jax.ShapeDtypeStruct((1, N_TOK), jnp.int32),
            jax.ShapeDtypeStruct((1, N_TOK), jnp.float32),
            jax.ShapeDtypeStruct((1, N_TOK), jnp.float32),
        ],
    )(router_scale, xf, gate_w)
    i0, i1, w0, w1 = out
    w0b = jnp.broadcast_to(w0.reshape(N_TOK, 1), (N_TOK, WL))
    w1b = jnp.broadcast_to(w1.reshape(N_TOK, 1), (N_TOK, WL))
    return i0[0], i1[0], w0b, w1b


def _dispatch(i0, i1):
    e_all = jnp.stack([i0, i1], axis=1).reshape(-1)
    masks = (e_all[None, :] == jnp.arange(E, dtype=e_all.dtype)[:, None])
    counts = jnp.sum(masks, axis=1)
    cums = jnp.cumsum(masks.astype(jnp.int32), axis=1)
    rank = jnp.sum(jnp.where(masks, cums, 0), axis=0) - 1
    padded = ((counts + BLK - 1) // BLK) * BLK
    ends = jnp.cumsum(padded)
    starts = ends - padded
    pos = (starts[e_all] + rank).astype(jnp.int32)
    bstart = jnp.arange(NB, dtype=jnp.int32) * BLK
    block_expert = jnp.sum(bstart[:, None] >= ends[None, :], axis=1)
    block_expert = jnp.minimum(block_expert, E - 1).astype(jnp.int32)
    nreal = (ends[-1] // BLK).astype(jnp.int32).reshape((1,))
    return pos[0::2], pos[1::2], block_expert, nreal


_SCH = 32


def _sc_scatter_body(xf_hbm, p0_hbm, p1_hbm, w0_hbm, w1_hbm,
                     out_hbm, ws_hbm,
                     p0v, p1v, wv0, wv1, rows_v, rsem, s0sem, s1sem, wssem):
    wid = lax.axis_index("s") * _NC + lax.axis_index("c")
    per_w = N_TOK // _NW
    n_chunks = per_w // _SCH
    base = wid * per_w
    rowbase = wid * n_chunks

    pltpu.sync_copy(p0_hbm.at[pl.ds(rowbase, n_chunks)], p0v)
    pltpu.sync_copy(p1_hbm.at[pl.ds(rowbase, n_chunks)], p1v)
    pltpu.sync_copy(w0_hbm.at[pl.ds(base, per_w)], wv0)
    pltpu.sync_copy(w1_hbm.at[pl.ds(base, per_w)], wv1)

    def fire(j, buf):
        pltpu.async_copy(xf_hbm.at[pl.ds(base + j * _SCH, _SCH)],
                         rows_v.at[buf], rsem.at[buf])

    fire(0, 0)

    def chunk(j, carry):
        buf = lax.rem(j, 2)
        nbuf = 1 - buf

        @pl.when(j + 1 < n_chunks)
        def _():
            @pl.when(j >= 1)
            def _():
                pltpu.make_async_copy(
                    rows_v.at[nbuf], out_hbm.at[pl.ds(base, _SCH)],
                    s0sem.at[nbuf]).wait()
                pltpu.make_async_copy(
                    rows_v.at[nbuf], out_hbm.at[pl.ds(base, _SCH)],
                    s1sem.at[nbuf]).wait()
            fire(j + 1, nbuf)

        pltpu.make_async_copy(
            xf_hbm.at[pl.ds(base, _SCH)], rows_v.at[buf],
            rsem.at[buf]).wait()
        pltpu.async_copy(rows_v.at[buf], out_hbm.at[p0v.at[j]],
                         s0sem.at[buf])
        pltpu.async_copy(rows_v.at[buf], out_hbm.at[p1v.at[j]],
                         s1sem.at[buf])
        pltpu.async_copy(wv0.at[pl.ds(j * _SCH, _SCH)],
                         ws_hbm.at[p0v.at[j]], wssem)
        pltpu.async_copy(wv1.at[pl.ds(j * _SCH, _SCH)],
                         ws_hbm.at[p1v.at[j]], wssem)
        return carry

    lax.fori_loop(0, n_chunks, chunk, 0)
    for slot in (0, 1):
        pltpu.make_async_copy(
            rows_v.at[slot], out_hbm.at[pl.ds(base, _SCH)],
            s0sem.at[slot]).wait()
        pltpu.make_async_copy(
            rows_v.at[slot], out_hbm.at[pl.ds(base, _SCH)],
            s1sem.at[slot]).wait()
    for _ in range(2 * (N_TOK // _NW // _SCH)):
        pltpu.make_async_copy(
            wv0.at[pl.ds(0, _SCH)], ws_hbm.at[pl.ds(base, _SCH)],
            wssem).wait()


def _sc_scatter(xf, pos0_2d, pos1_2d, w0b, w1b):
    mesh = plsc.VectorSubcoreMesh(core_axis_name="c", subcore_axis_name="s")
    per_w = N_TOK // _NW
    n_chunks = per_w // _SCH
    f = pl.kernel(
        _sc_scatter_body,
        out_type=[
            jax.ShapeDtypeStruct((G, DIM), jnp.float32),
            jax.ShapeDtypeStruct((G, WL), jnp.float32),
        ],
        mesh=mesh,
        scratch_types=[
            pltpu.VMEM((n_chunks, _SCH), jnp.int32),
            pltpu.VMEM((n_chunks, _SCH), jnp.int32),
            pltpu.VMEM((per_w, WL), jnp.float32),
            pltpu.VMEM((per_w, WL), jnp.float32),
            pltpu.VMEM((2, _SCH, DIM), jnp.float32),
            pltpu.SemaphoreType.DMA((2,)),
            pltpu.SemaphoreType.DMA((2,)),
            pltpu.SemaphoreType.DMA((2,)),
            pltpu.SemaphoreType.DMA,
        ],
    )
    return f(xf, pos0_2d, pos1_2d, w0b, w1b)


def _ffn_body(be_ref, nr_ref, x_ref, w1_ref, b1_ref, w2_ref, b2_ref,
              w3_ref, b3_ref, ws_ref, o_ref):
    b = pl.program_id(0)

    @pl.when(b < nr_ref[0])
    def _():
        x = x_ref[...]
        h1 = lax.dot_general(x, w1_ref[0], (((1,), (1,)), ((), ())),
                             preferred_element_type=jnp.float32)
        h1 = h1 + b1_ref[0]
        h2 = lax.dot_general(x, w2_ref[0], (((1,), (1,)), ((), ())),
                             preferred_element_type=jnp.float32)
        h2 = h2 + b2_ref[0]
        h = (h1 * jax.nn.sigmoid(h1)) * h2
        o = lax.dot_general(h, w3_ref[0], (((1,), (1,)), ((), ())),
                            preferred_element_type=jnp.float32)
        o = o + b3_ref[0]
        o_ref[...] = o * ws_ref[:, :1]

def _ffn(x_sorted, ws_sorted, block_expert, nreal, W1, b1, W2, b2, W3, b3):
    spec = pltpu.PrefetchScalarGridSpec(
        num_scalar_prefetch=2,
        grid=(NB,),
        in_specs=[
            pl.BlockSpec((BLK, DIM), lambda b, be, nr: (b, 0)),
            pl.BlockSpec((1, HIDDEN, DIM), lambda b, be, nr: (be[b], 0, 0)),
            pl.BlockSpec((1, 1, HIDDEN), lambda b, be, nr: (be[b], 0, 0)),
            pl.BlockSpec((1, HIDDEN, DIM), lambda b, be, nr: (be[b], 0, 0)),
            pl.BlockSpec((1, 1, HIDDEN), lambda b, be, nr: (be[b], 0, 0)),
            pl.BlockSpec((1, DIM, HIDDEN), lambda b, be, nr: (be[b], 0, 0)),
            pl.BlockSpec((1, 1, DIM), lambda b, be, nr: (be[b], 0, 0)),
            pl.BlockSpec((BLK, WL), lambda b, be, nr: (b, 0)),
        ],
        out_specs=pl.BlockSpec((BLK, DIM), lambda b, be, nr: (b, 0)),
    )
    return pl.pallas_call(
        _ffn_body,
        grid_spec=spec,
        out_shape=jax.ShapeDtypeStruct((G, DIM), jnp.float32),
        compiler_params=pltpu.CompilerParams(
            dimension_semantics=("arbitrary",)),
    )(block_expert, nreal, x_sorted,
      W1, b1.reshape(E, 1, HIDDEN), W2, b2.reshape(E, 1, HIDDEN),
      W3, b3.reshape(E, 1, DIM), ws_sorted)


_CCH = 16


def _sc_combine_body(o_hbm, p0_hbm, p1_hbm, y_hbm,
                     i0v, i1v, r0v, r1v, gsem, wsem):
    wid = lax.axis_index("s") * _NC + lax.axis_index("c")
    per_w = N_TOK // _NW
    n_chunks = per_w // _CCH
    base = wid * per_w
    rowbase = wid * n_chunks

    pltpu.sync_copy(p0_hbm.at[pl.ds(rowbase, n_chunks)], i0v)
    pltpu.sync_copy(p1_hbm.at[pl.ds(rowbase, n_chunks)], i1v)

    def fire(j, buf):
        pltpu.async_copy(o_hbm.at[i0v.at[j]], r0v.at[buf], gsem.at[buf])
        pltpu.async_copy(o_hbm.at[i1v.at[j]], r1v.at[buf], gsem.at[buf])

    fire(0, 0)

    def chunk(j, carry):
        buf = lax.rem(j, 2)
        nbuf = 1 - buf

        @pl.when(j + 1 < n_chunks)
        def _():
            @pl.when(j >= 1)
            def _():
                pltpu.make_async_copy(
                    r0v.at[nbuf], y_hbm.at[pl.ds(base, _CCH)],
                    wsem.at[nbuf]).wait()
            fire(j + 1, nbuf)

        pltpu.make_async_copy(
            o_hbm.at[i0v.at[j]], r0v.at[buf], gsem.at[buf]).wait()
        pltpu.make_async_copy(
            o_hbm.at[i1v.at[j]], r1v.at[buf], gsem.at[buf]).wait()

        def row(t, c2):
            for c in range(DIM // 16):
                s = pl.ds(c * 16, 16)
                plsc.addupdate(r0v.at[buf, t, s], r1v[buf, t, s])
            return c2

        lax.fori_loop(0, _CCH, row, 0)
        pltpu.async_copy(r0v.at[buf], y_hbm.at[pl.ds(base + j * _CCH, _CCH)],
                         wsem.at[buf])
        return carry

    lax.fori_loop(0, n_chunks, chunk, 0)
    for slot in (0, 1):
        pltpu.make_async_copy(
            r0v.at[slot], y_hbm.at[pl.ds(base, _CCH)],
            wsem.at[slot]).wait()


def _sc_combine(out_sorted, pos0_2d, pos1_2d):
    mesh = plsc.VectorSubcoreMesh(core_axis_name="c", subcore_axis_name="s")
    n_chunks = (N_TOK // _NW) // _CCH
    f = pl.kernel(
        _sc_combine_body,
        out_type=jax.ShapeDtypeStruct((N_TOK, DIM), jnp.float32),
        mesh=mesh,
        scratch_types=[
            pltpu.VMEM((n_chunks, _CCH), jnp.int32),
            pltpu.VMEM((n_chunks, _CCH), jnp.int32),
            pltpu.VMEM((2, _CCH, DIM), jnp.float32),
            pltpu.VMEM((2, _CCH, DIM), jnp.float32),
            pltpu.SemaphoreType.DMA((2,)),
            pltpu.SemaphoreType.DMA((2,)),
        ],
    )
    return f(out_sorted, pos0_2d, pos1_2d)


def kernel(x, gate_w, router_scale, W1, b1, W2, b2, W3, b3):
    Bs, Ts, C = x.shape
    xf = x.reshape(Bs * Ts, C)
    i0, i1, w0b, w1b = _router(xf, gate_w, router_scale)
    pos0, pos1, block_expert, nreal = _dispatch(i0, i1)
    x_sorted, ws_sorted = _sc_scatter(
        xf, pos0.reshape(-1, _SCH), pos1.reshape(-1, _SCH), w0b, w1b)
    out_sorted = _ffn(x_sorted, ws_sorted, block_expert, nreal,
                      W1, b1, W2, b2, W3, b3)
    y = _sc_combine(out_sorted, pos0.reshape(-1, _CCH), pos1.reshape(-1, _CCH))
    return y.reshape(Bs, Ts, C)


# speedup vs baseline: 1.6521x; 1.6521x over previous
import jax
import jax.numpy as jnp
from jax import lax
from jax.experimental import pallas as pl
from jax.experimental.pallas import tpu as pltpu
from jax.experimental.pallas import tpu_sc as plsc

DIM = 1024
E = 8
HIDDEN = 2048
N_TOK = 4096
BLK = 512
A = N_TOK * 2
G = A + E * BLK
NB = G // BLK
WL = 128

_NC, _NS = 2, 16
_NW = _NC * _NS


def _router_body(rs_ref, x_ref, gw_ref, i0_ref, i1_ref, w0_ref, w1_ref):
    xf = x_ref[...]
    gw = gw_ref[...]
    logits = lax.dot_general(gw, xf, (((1,), (1,)), ((), ())),
                             preferred_element_type=jnp.float32)
    scaled = logits * rs_ref[0]
    iota = lax.broadcasted_iota(jnp.int32, scaled.shape, 0)
    m0 = jnp.max(scaled, axis=0, keepdims=True)
    i0 = jnp.min(jnp.where(scaled == m0, iota, E), axis=0, keepdims=True)
    masked = jnp.where(iota == i0, -jnp.inf, scaled)
    m1 = jnp.max(masked, axis=0, keepdims=True)
    i1 = jnp.min(jnp.where(masked == m1, iota, E), axis=0, keepdims=True)
    e1 = jnp.exp(m1 - m0)
    w0 = 1.0 / (1.0 + e1)
    w1 = e1 / (1.0 + e1)
    i0_ref[...] = i0
    i1_ref[...] = i1
    w0_ref[...] = w0
    w1_ref[...] = w1


def _router(xf, gate_w, router_scale):
    out = pl.pallas_call(
        _router_body,
        in_specs=[
            pl.BlockSpec(memory_space=pltpu.SMEM),
            pl.BlockSpec(memory_space=pltpu.VMEM),
            pl.BlockSpec(memory_space=pltpu.VMEM),
        ],
        out_specs=[pl.BlockSpec(memory_space=pltpu.VMEM)] * 4,
        out_shape=[
            jax.ShapeDtypeStruct((1, N_TOK), jnp.int32),
            jax.ShapeDtypeStruct((1, N_TOK), jnp.int32),
            jax.ShapeDtypeStruct((1, N_TOK), jnp.float32),
            jax.ShapeDtypeStruct((1, N_TOK), jnp.float32),
        ],
    )(router_scale, xf, gate_w)
    i0, i1, w0, w1 = out
    w0b = jnp.broadcast_to(w0.reshape(N_TOK, 1), (N_TOK, WL))
    w1b = jnp.broadcast_to(w1.reshape(N_TOK, 1), (N_TOK, WL))
    return i0[0], i1[0], w0b, w1b


def _dispatch(i0, i1):
    e_all = jnp.stack([i0, i1], axis=1).reshape(-1)
    masks = (e_all[None, :] == jnp.arange(E, dtype=e_all.dtype)[:, None])
    counts = jnp.sum(masks, axis=1)
    cums = jnp.cumsum(masks.astype(jnp.int32), axis=1)
    rank = jnp.sum(jnp.where(masks, cums, 0), axis=0) - 1
    padded = ((counts + BLK - 1) // BLK) * BLK
    ends = jnp.cumsum(padded)
    starts = ends - padded
    pos = (starts[e_all] + rank).astype(jnp.int32)
    bstart = jnp.arange(NB, dtype=jnp.int32) * BLK
    block_expert = jnp.sum(bstart[:, None] >= ends[None, :], axis=1)
    block_expert = jnp.minimum(block_expert, E - 1).astype(jnp.int32)
    nreal = (ends[-1] // BLK).astype(jnp.int32).reshape((1,))
    return pos[0::2], pos[1::2], block_expert, nreal


_SCH = 32


def _sc_scatter_body(xf_hbm, p0_hbm, p1_hbm, w0_hbm, w1_hbm,
                     out_hbm, ws_hbm,
                     p0v, p1v, wv0, wv1, rows_v, rsem, s0sem, s1sem, wssem):
    wid = lax.axis_index("s") * _NC + lax.axis_index("c")
    per_w = N_TOK // _NW
    n_chunks = per_w // _SCH
    base = wid * per_w
    rowbase = wid * n_chunks

    pltpu.sync_copy(p0_hbm.at[pl.ds(rowbase, n_chunks)], p0v)
    pltpu.sync_copy(p1_hbm.at[pl.ds(rowbase, n_chunks)], p1v)
    pltpu.sync_copy(w0_hbm.at[pl.ds(base, per_w)], wv0)
    pltpu.sync_copy(w1_hbm.at[pl.ds(base, per_w)], wv1)

    def fire(j, buf):
        pltpu.async_copy(xf_hbm.at[pl.ds(base + j * _SCH, _SCH)],
                         rows_v.at[buf], rsem.at[buf])

    fire(0, 0)

    def chunk(j, carry):
        buf = lax.rem(j, 2)
        nbuf = 1 - buf

        @pl.when(j + 1 < n_chunks)
        def _():
            @pl.when(j >= 1)
            def _():
                pltpu.make_async_copy(
                    rows_v.at[nbuf], out_hbm.at[pl.ds(base, _SCH)],
                    s0sem.at[nbuf]).wait()
                pltpu.make_async_copy(
                    rows_v.at[nbuf], out_hbm.at[pl.ds(base, _SCH)],
                    s1sem.at[nbuf]).wait()
            fire(j + 1, nbuf)

        pltpu.make_async_copy(
            xf_hbm.at[pl.ds(base, _SCH)], rows_v.at[buf],
            rsem.at[buf]).wait()
        pltpu.async_copy(rows_v.at[buf], out_hbm.at[p0v.at[j]],
                         s0sem.at[buf])
        pltpu.async_copy(rows_v.at[buf], out_hbm.at[p1v.at[j]],
                         s1sem.at[buf])
        pltpu.async_copy(wv0.at[pl.ds(j * _SCH, _SCH)],
                         ws_hbm.at[p0v.at[j]], wssem)
        pltpu.async_copy(wv1.at[pl.ds(j * _SCH, _SCH)],
                         ws_hbm.at[p1v.at[j]], wssem)
        return carry

    lax.fori_loop(0, n_chunks, chunk, 0)
    for slot in (0, 1):
        pltpu.make_async_copy(
            rows_v.at[slot], out_hbm.at[pl.ds(base, _SCH)],
            s0sem.at[slot]).wait()
        pltpu.make_async_copy(
            rows_v.at[slot], out_hbm.at[pl.ds(base, _SCH)],
            s1sem.at[slot]).wait()
    for _ in range(2 * (N_TOK // _NW // _SCH)):
        pltpu.make_async_copy(
            wv0.at[pl.ds(0, _SCH)], ws_hbm.at[pl.ds(base, _SCH)],
            wssem).wait()


def _sc_scatter(xf, pos0_2d, pos1_2d, w0b, w1b):
    mesh = plsc.VectorSubcoreMesh(core_axis_name="c", subcore_axis_name="s")
    per_w = N_TOK // _NW
    n_chunks = per_w // _SCH
    f = pl.kernel(
        _sc_scatter_body,
        out_type=[
            jax.ShapeDtypeStruct((G, DIM), jnp.float32),
            jax.ShapeDtypeStruct((G, WL), jnp.float32),
        ],
        mesh=mesh,
        scratch_types=[
            pltpu.VMEM((n_chunks, _SCH), jnp.int32),
            pltpu.VMEM((n_chunks, _SCH), jnp.int32),
            pltpu.VMEM((per_w, WL), jnp.float32),
            pltpu.VMEM((per_w, WL), jnp.float32),
            pltpu.VMEM((2, _SCH, DIM), jnp.float32),
            pltpu.SemaphoreType.DMA((2,)),
            pltpu.SemaphoreType.DMA((2,)),
            pltpu.SemaphoreType.DMA((2,)),
            pltpu.SemaphoreType.DMA,
        ],
    )
    return f(xf, pos0_2d, pos1_2d, w0b, w1b)


def _ffn_body(be_ref, nr_ref, x_ref, w1_ref, b1_ref, w2_ref, b2_ref,
              w3_ref, b3_ref, ws_ref, o_ref):
    b = pl.program_id(0)

    @pl.when(b < nr_ref[0])
    def _():
        x = x_ref[...]
        h1 = lax.dot_general(x, w1_ref[0], (((1,), (1,)), ((), ())),
                             preferred_element_type=jnp.float32)
        h1 = h1 + b1_ref[0]
        h2 = lax.dot_general(x, w2_ref[0], (((1,), (1,)), ((), ())),
                             preferred_element_type=jnp.float32)
        h2 = h2 + b2_ref[0]
        h = (h1 * jax.nn.sigmoid(h1)) * h2
        o = lax.dot_general(h, w3_ref[0], (((1,), (1,)), ((), ())),
                            preferred_element_type=jnp.float32)
        o = o + b3_ref[0]
        o_ref[...] = o * ws_ref[:, :1]

def _ffn(x_sorted, ws_sorted, block_expert, nreal, W1, b1, W2, b2, W3, b3):
    spec = pltpu.PrefetchScalarGridSpec(
        num_scalar_prefetch=2,
        grid=(NB,),
        in_specs=[
            pl.BlockSpec((BLK, DIM), lambda b, be, nr: (b, 0)),
            pl.BlockSpec((1, HIDDEN, DIM), lambda b, be, nr: (be[b], 0, 0)),
            pl.BlockSpec((1, 1, HIDDEN), lambda b, be, nr: (be[b], 0, 0)),
            pl.BlockSpec((1, HIDDEN, DIM), lambda b, be, nr: (be[b], 0, 0)),
            pl.BlockSpec((1, 1, HIDDEN), lambda b, be, nr: (be[b], 0, 0)),
            pl.BlockSpec((1, DIM, HIDDEN), lambda b, be, nr: (be[b], 0, 0)),
            pl.BlockSpec((1, 1, DIM), lambda b, be, nr: (be[b], 0, 0)),
            pl.BlockSpec((BLK, WL), lambda b, be, nr: (b, 0)),
        ],
        out_specs=pl.BlockSpec((BLK, DIM), lambda b, be, nr: (b, 0)),
    )
    return pl.pallas_call(
        _ffn_body,
        grid_spec=spec,
        out_shape=jax.ShapeDtypeStruct((G, DIM), jnp.float32),
        compiler_params=pltpu.CompilerParams(
            dimension_semantics=("arbitrary",),
            vmem_limit_bytes=110 * 1024 * 1024),
    )(block_expert, nreal, x_sorted,
      W1, b1.reshape(E, 1, HIDDEN), W2, b2.reshape(E, 1, HIDDEN),
      W3, b3.reshape(E, 1, DIM), ws_sorted)


_CCH = 16


def _sc_combine_body(o_hbm, p0_hbm, p1_hbm, y_hbm,
                     i0v, i1v, r0v, r1v, gsem, wsem):
    wid = lax.axis_index("s") * _NC + lax.axis_index("c")
    per_w = N_TOK // _NW
    n_chunks = per_w // _CCH
    base = wid * per_w
    rowbase = wid * n_chunks

    pltpu.sync_copy(p0_hbm.at[pl.ds(rowbase, n_chunks)], i0v)
    pltpu.sync_copy(p1_hbm.at[pl.ds(rowbase, n_chunks)], i1v)

    def fire(j, buf):
        pltpu.async_copy(o_hbm.at[i0v.at[j]], r0v.at[buf], gsem.at[buf])
        pltpu.async_copy(o_hbm.at[i1v.at[j]], r1v.at[buf], gsem.at[buf])

    fire(0, 0)

    def chunk(j, carry):
        buf = lax.rem(j, 2)
        nbuf = 1 - buf

        @pl.when(j + 1 < n_chunks)
        def _():
            @pl.when(j >= 1)
            def _():
                pltpu.make_async_copy(
                    r0v.at[nbuf], y_hbm.at[pl.ds(base, _CCH)],
                    wsem.at[nbuf]).wait()
            fire(j + 1, nbuf)

        pltpu.make_async_copy(
            o_hbm.at[i0v.at[j]], r0v.at[buf], gsem.at[buf]).wait()
        pltpu.make_async_copy(
            o_hbm.at[i1v.at[j]], r1v.at[buf], gsem.at[buf]).wait()

        def row(t, c2):
            for c in range(DIM // 16):
                s = pl.ds(c * 16, 16)
                plsc.addupdate(r0v.at[buf, t, s], r1v[buf, t, s])
            return c2

        lax.fori_loop(0, _CCH, row, 0)
        pltpu.async_copy(r0v.at[buf], y_hbm.at[pl.ds(base + j * _CCH, _CCH)],
                         wsem.at[buf])
        return carry

    lax.fori_loop(0, n_chunks, chunk, 0)
    for slot in (0, 1):
        pltpu.make_async_copy(
            r0v.at[slot], y_hbm.at[pl.ds(base, _CCH)],
            wsem.at[slot]).wait()


def _sc_combine(out_sorted, pos0_2d, pos1_2d):
    mesh = plsc.VectorSubcoreMesh(core_axis_name="c", subcore_axis_name="s")
    n_chunks = (N_TOK // _NW) // _CCH
    f = pl.kernel(
        _sc_combine_body,
        out_type=jax.ShapeDtypeStruct((N_TOK, DIM), jnp.float32),
        mesh=mesh,
        scratch_types=[
            pltpu.VMEM((n_chunks, _CCH), jnp.int32),
            pltpu.VMEM((n_chunks, _CCH), jnp.int32),
            pltpu.VMEM((2, _CCH, DIM), jnp.float32),
            pltpu.VMEM((2, _CCH, DIM), jnp.float32),
            pltpu.SemaphoreType.DMA((2,)),
            pltpu.SemaphoreType.DMA((2,)),
        ],
    )
    return f(out_sorted, pos0_2d, pos1_2d)


def kernel(x, gate_w, router_scale, W1, b1, W2, b2, W3, b3):
    Bs, Ts, C = x.shape
    xf = x.reshape(Bs * Ts, C)
    i0, i1, w0b, w1b = _router(xf, gate_w, router_scale)
    pos0, pos1, block_expert, nreal = _dispatch(i0, i1)
    x_sorted, ws_sorted = _sc_scatter(
        xf, pos0.reshape(-1, _SCH), pos1.reshape(-1, _SCH), w0b, w1b)
    out_sorted = _ffn(x_sorted, ws_sorted, block_expert, nreal,
                      W1, b1, W2, b2, W3, b3)
    y = _sc_combine(out_sorted, pos0.reshape(-1, _CCH), pos1.reshape(-1, _CCH))
    return y.reshape(Bs, Ts, C)
